# baseline (speedup 1.0000x reference)
import functools

import jax
import jax.numpy as jnp
from jax import lax
from jax.experimental import pallas as pl
from jax.experimental.pallas import tpu as pltpu
from jax.experimental.pallas import tpu_sc as plsc

EPS = 1e-5

NC = 2
NS = 16
K = 40


def _node_pre(h, Ws, Wd):
    N, D = h.shape
    H = Ws.shape[1]
    BN = 2000

    def body(h_ref, ws_ref, wd_ref, a_ref, b_ref):
        hb = h_ref[...]
        a_ref[...] = jnp.dot(hb, ws_ref[...], preferred_element_type=jnp.float32)
        b_ref[...] = jnp.dot(hb, wd_ref[...], preferred_element_type=jnp.float32)

    return pl.pallas_call(
        body,
        grid=(N // BN,),
        in_specs=[
            pl.BlockSpec((BN, D), lambda i: (i, 0)),
            pl.BlockSpec((D, H), lambda i: (0, 0)),
            pl.BlockSpec((D, H), lambda i: (0, 0)),
        ],
        out_specs=[
            pl.BlockSpec((BN, H), lambda i: (i, 0)),
            pl.BlockSpec((BN, H), lambda i: (i, 0)),
        ],
        out_shape=[jax.ShapeDtypeStruct((N, H), jnp.float32)] * 2,
    )(h, Ws, Wd)


def _edge_pre(ea, We, b):
    E, ED = ea.shape
    H = We.shape[1]
    BE = 8000

    def body(ea_ref, we_ref, b_ref, c_ref):
        c_ref[...] = (
            jnp.dot(ea_ref[...], we_ref[...], preferred_element_type=jnp.float32)
            + b_ref[...]
        )

    return pl.pallas_call(
        body,
        grid=(E // BE,),
        in_specs=[
            pl.BlockSpec((BE, ED), lambda i: (i, 0)),
            pl.BlockSpec((ED, H), lambda i: (0, 0)),
            pl.BlockSpec((1, H), lambda i: (0, 0)),
        ],
        out_specs=pl.BlockSpec((BE, H), lambda i: (i, 0)),
        out_shape=jax.ShapeDtypeStruct((E, H), jnp.float32),
    )(ea, We, b.reshape(1, H))


def _edge_msg(src, dst, A, B, C):
    N, H = A.shape
    E = src.shape[0]
    EW = E // (NC * NS)
    NCHUNK = EW // K
    RPT = (N // NS) // 8 * 8
    TAIL = N - NS * RPT
    mesh = plsc.VectorSubcoreMesh(core_axis_name="c", subcore_axis_name="s")

    @functools.partial(
        pl.kernel,
        out_type=[
            jax.ShapeDtypeStruct((NC, N, H), jnp.float32),
            jax.ShapeDtypeStruct((NC, N, H), jnp.float32),
        ],
        mesh=mesh,
        scratch_types=[
            pltpu.VMEM((K,), jnp.int32),
            pltpu.VMEM((K,), jnp.int32),
            pltpu.VMEM((K, H), jnp.float32),
            pltpu.VMEM((K, H), jnp.float32),
            pltpu.VMEM((K, H), jnp.float32),
            pltpu.VMEM((K, H), jnp.float32),
            pltpu.VMEM((K,), jnp.int32),
            pltpu.VMEM((K,), jnp.int32),
            pltpu.VMEM((K, H), jnp.float32),
            pltpu.VMEM((K, H), jnp.float32),
            pltpu.VMEM((K, H), jnp.float32),
            pltpu.VMEM((K, H), jnp.float32),
            pltpu.VMEM_SHARED((N, H), jnp.float32),
            pltpu.SemaphoreType.DMA,
            pltpu.SemaphoreType.DMA,
            pltpu.SemaphoreType.DMA,
            pltpu.SemaphoreType.DMA,
            pltpu.SemaphoreType.DMA,
            pltpu.SemaphoreType.DMA,
            pltpu.SemaphoreType.DMA,
            pltpu.SemaphoreType.DMA,
        ],
    )
    def body(src_hbm, dst_hbm, a_hbm, b_hbm, c_hbm, s_out, deg_out,
             is0, id0, ra0, rb0, rc0, rm0,
             is1, id1, ra1, rb1, rc1, rm1, s_sh,
             semA0, semB0, semC0, semS0, semA1, semB1, semC1, semS1):
        c = lax.axis_index("c")
        s = lax.axis_index("s")
        z16 = jnp.zeros((16,), jnp.float32)
        sets = [
            (is0, id0, ra0, rb0, rc0, rm0, semA0, semB0, semC0, semS0),
            (is1, id1, ra1, rb1, rc1, rm1, semA1, semB1, semC1, semS1),
        ]
        nb = s * RPT
        nfull = RPT // K
        rem = RPT - nfull * K

        def zero_accumulator(zbuf):
            def zero_row(i, _):
                for j in range(H // 16):
                    zbuf[i, pl.ds(j * 16, 16)] = z16
                return 0

            lax.fori_loop(0, K, zero_row, 0)

            def zero_copy(q, _):
                pltpu.sync_copy(zbuf, s_sh.at[pl.ds(nb + q * K, K)])
                return 0

            lax.fori_loop(0, nfull, zero_copy, 0)
            if rem:
                pltpu.sync_copy(zbuf.at[pl.ds(0, rem)],
                                s_sh.at[pl.ds(nb + nfull * K, rem)])
            if TAIL:
                @pl.when(s == NS - 1)
                def _zero_tail():
                    pltpu.sync_copy(zbuf.at[pl.ds(0, TAIL)],
                                    s_sh.at[pl.ds(NS * RPT, TAIL)])

        def write_out(out_hbm):
            pltpu.sync_copy(s_sh.at[pl.ds(nb, RPT)], out_hbm.at[c, pl.ds(nb, RPT)])
            if TAIL:
                @pl.when(s == NS - 1)
                def _out_tail():
                    pltpu.sync_copy(s_sh.at[pl.ds(NS * RPT, TAIL)],
                                    out_hbm.at[c, pl.ds(NS * RPT, TAIL)])

        zero_accumulator(ra0)
        plsc.subcore_barrier()

        base_e = (c * NS + s) * EW

        def start_gathers(b, g):
            isb, idb, rab, rbb, rcb, rmb, sA, sB, sC, sS = sets[b]
            eb = base_e + g * K
            pltpu.sync_copy(src_hbm.at[pl.ds(eb, K)], isb)
            pltpu.sync_copy(dst_hbm.at[pl.ds(eb, K)], idb)
            pltpu.async_copy(a_hbm.at[isb], rab, sA)
            pltpu.async_copy(b_hbm.at[idb], rbb, sB)
            pltpu.async_copy(c_hbm.at[pl.ds(eb, K)], rcb, sC)

        def finish_chunk(b, g):
            isb, idb, rab, rbb, rcb, rmb, sA, sB, sC, sS = sets[b]
            pltpu.make_async_copy(a_hbm.at[isb], rab, sA).wait()
            pltpu.make_async_copy(b_hbm.at[idb], rbb, sB).wait()
            pltpu.make_async_copy(c_hbm.at[pl.ds(0, K)], rcb, sC).wait()

            @plsc.parallel_loop(0, K, unroll=2)
            def row(i):
                for j in range(H // 16):
                    sl = pl.ds(j * 16, 16)
                    rmb[i, sl] = jnp.maximum(rab[i, sl] + rbb[i, sl] + rcb[i, sl], 0.0)

            pltpu.sync_copy(rmb, s_sh.at[idb], add=True)

            @pl.when(g + 2 < NCHUNK)
            def _prefetch():
                start_gathers(b, g + 2)

        start_gathers(0, 0)
        start_gathers(1, 1)

        def pair(gp, _):
            finish_chunk(0, 2 * gp)
            finish_chunk(1, 2 * gp + 1)
            return 0

        lax.fori_loop(0, NCHUNK // 2, pair, 0)

        plsc.subcore_barrier()
        write_out(s_out)
        plsc.subcore_barrier()

        zero_accumulator(rm0)
        plsc.subcore_barrier()

        e0 = jnp.where(lax.iota(jnp.int32, 16) == 0, 1.0, 0.0).astype(jnp.float32)

        def ones_row(i, _):
            rm0[i, pl.ds(0, 16)] = e0
            return 0

        lax.fori_loop(0, K, ones_row, 0)

        def deg_half(b, gp, g):
            idxb = is0 if b == 0 else is1
            sS = semS0 if b == 0 else semS1

            @pl.when(gp > 0)
            def _drain():
                pltpu.make_async_copy(rm0, s_sh.at[idxb], sS).wait()

            pltpu.sync_copy(dst_hbm.at[pl.ds(base_e + g * K, K)], idxb)
            pltpu.async_copy(rm0, s_sh.at[idxb], sS, add=True)

        def deg_pair(gp, _):
            deg_half(0, gp, 2 * gp)
            deg_half(1, gp, 2 * gp + 1)
            return 0

        lax.fori_loop(0, NCHUNK // 2, deg_pair, 0)
        pltpu.make_async_copy(rm0, s_sh.at[is0], semS0).wait()
        pltpu.make_async_copy(rm0, s_sh.at[is1], semS1).wait()

        plsc.subcore_barrier()
        write_out(deg_out)

    return body(src, dst, A, B, C)


def _node_post(h, S, deg, W2m, b2m, W1uh, W1ua, b1u, W2u, b2u, ln_g, ln_b):
    N, D = h.shape
    H = W2m.shape[0]
    BN = 2000

    def body(h_ref, s_ref, d_ref, w2m_ref, b2m_ref, w1uh_ref, w1ua_ref,
             b1u_ref, w2u_ref, b2u_ref, lng_ref, lnb_ref, out_ref):
        hb = h_ref[...]
        Sb = s_ref[0] + s_ref[1]
        degb = d_ref[0, :, 0:1] + d_ref[1, :, 0:1]
        agg = (
            jnp.dot(Sb, w2m_ref[...], preferred_element_type=jnp.float32)
            + degb * b2m_ref[...]
        )
        y = (
            jnp.dot(hb, w1uh_ref[...], preferred_element_type=jnp.float32)
            + jnp.dot(agg, w1ua_ref[...], preferred_element_type=jnp.float32)
            + b1u_ref[...]
        )
        y = jnp.maximum(y, 0.0)
        hn = jnp.dot(y, w2u_ref[...], preferred_element_type=jnp.float32) + b2u_ref[...]
        z = hb + hn
        mean = jnp.mean(z, axis=1, keepdims=True)
        zc = z - mean
        var = jnp.mean(zc * zc, axis=1, keepdims=True)
        out_ref[...] = zc * lax.rsqrt(var + EPS) * lng_ref[...] + lnb_ref[...]

    full = lambda r, c: pl.BlockSpec((r, c), lambda i: (0, 0))
    return pl.pallas_call(
        body,
        grid=(N // BN,),
        in_specs=[
            pl.BlockSpec((BN, D), lambda i: (i, 0)),
            pl.BlockSpec((NC, BN, H), lambda i: (0, i, 0)),
            pl.BlockSpec((NC, BN, H), lambda i: (0, i, 0)),
            full(H, D), full(1, D), full(D, H), full(H, H), full(1, H),
            full(H, D), full(1, D), full(1, D), full(1, D),
        ],
        out_specs=pl.BlockSpec((BN, D), lambda i: (i, 0)),
        out_shape=jax.ShapeDtypeStruct((N, D), jnp.float32),
    )(h, S, deg, W2m, b2m.reshape(1, -1), W1uh, W1ua, b1u.reshape(1, -1),
      W2u, b2u.reshape(1, -1), ln_g.reshape(1, -1), ln_b.reshape(1, -1))


def kernel(h, edge_index, edge_attr, num_nodes, W1m, b1m, g1m, be1m, W2m, b2m,
           W1u, b1u, g1u, be1u, W2u, b2u, ln_g, ln_b):
    N, D = h.shape
    H = W1m.shape[1]

    s1m = g1m * (1.0 / jnp.sqrt(1.0 + EPS))
    Wsrc = W1m[:D] * s1m
    Wdst = W1m[D:2 * D] * s1m
    We = W1m[2 * D:] * s1m
    b1m_f = b1m * s1m + be1m
    s1u = g1u * (1.0 / jnp.sqrt(1.0 + EPS))
    W1uh = W1u[:D] * s1u
    W1ua = W1u[D:] * s1u
    b1u_f = b1u * s1u + be1u

    src = edge_index[0]
    dst = edge_index[1]

    A, B = _node_pre(h, Wsrc, Wdst)
    C = _edge_pre(edge_attr, We, b1m_f)
    S, deg = _edge_msg(src, dst, A, B, C)
    return _node_post(h, S, deg, W2m, b2m, W1uh, W1ua, b1u_f, W2u, b2u, ln_g, ln_b)


# speedup vs baseline: 1.0655x; 1.0655x over previous
import functools

import jax
import jax.numpy as jnp
from jax import lax
from jax.experimental import pallas as pl
from jax.experimental.pallas import tpu as pltpu
from jax.experimental.pallas import tpu_sc as plsc

EPS = 1e-5

NC = 2
NS = 16
K = 40


def _node_pre(h, Ws, Wd):
    N, D = h.shape
    H = Ws.shape[1]
    BN = 2000

    def body(h_ref, ws_ref, wd_ref, a_ref, b_ref):
        hb = h_ref[...]
        a_ref[...] = jnp.dot(hb, ws_ref[...], preferred_element_type=jnp.float32)
        b_ref[...] = jnp.dot(hb, wd_ref[...], preferred_element_type=jnp.float32)

    return pl.pallas_call(
        body,
        grid=(N // BN,),
        in_specs=[
            pl.BlockSpec((BN, D), lambda i: (i, 0)),
            pl.BlockSpec((D, H), lambda i: (0, 0)),
            pl.BlockSpec((D, H), lambda i: (0, 0)),
        ],
        out_specs=[
            pl.BlockSpec((BN, H), lambda i: (i, 0)),
            pl.BlockSpec((BN, H), lambda i: (i, 0)),
        ],
        out_shape=[jax.ShapeDtypeStruct((N, H), jnp.float32)] * 2,
    )(h, Ws, Wd)


def _edge_pre(ea, We, b):
    E, ED = ea.shape
    H = We.shape[1]
    BE = 8000

    def body(ea_ref, we_ref, b_ref, c_ref):
        c_ref[...] = (
            jnp.dot(ea_ref[...], we_ref[...], preferred_element_type=jnp.float32)
            + b_ref[...]
        )

    return pl.pallas_call(
        body,
        grid=(E // BE,),
        in_specs=[
            pl.BlockSpec((BE, ED), lambda i: (i, 0)),
            pl.BlockSpec((ED, H), lambda i: (0, 0)),
            pl.BlockSpec((1, H), lambda i: (0, 0)),
        ],
        out_specs=pl.BlockSpec((BE, H), lambda i: (i, 0)),
        out_shape=jax.ShapeDtypeStruct((E, H), jnp.float32),
    )(ea, We, b.reshape(1, H))


def _edge_msg(src, dst, A, B, C):
    N, H = A.shape
    E = src.shape[0]
    EW = E // (NC * NS)
    NCHUNK = EW // K
    RPT = (N // NS) // 8 * 8
    TAIL = N - NS * RPT
    mesh = plsc.VectorSubcoreMesh(core_axis_name="c", subcore_axis_name="s")

    @functools.partial(
        pl.kernel,
        out_type=[jax.ShapeDtypeStruct((NC, N, H), jnp.float32)],
        mesh=mesh,
        scratch_types=[
            pltpu.VMEM((K,), jnp.int32),
            pltpu.VMEM((K,), jnp.int32),
            pltpu.VMEM((K, H), jnp.float32),
            pltpu.VMEM((K, H), jnp.float32),
            pltpu.VMEM((K, H), jnp.float32),
            pltpu.VMEM((K,), jnp.int32),
            pltpu.VMEM((K,), jnp.int32),
            pltpu.VMEM((K, H), jnp.float32),
            pltpu.VMEM((K, H), jnp.float32),
            pltpu.VMEM((K, H), jnp.float32),
            pltpu.VMEM((K,), jnp.int32),
            pltpu.VMEM((K,), jnp.int32),
            pltpu.VMEM((K, H), jnp.float32),
            pltpu.VMEM((K, H), jnp.float32),
            pltpu.VMEM((K, H), jnp.float32),
            pltpu.VMEM_SHARED((N, H), jnp.float32),
            pltpu.SemaphoreType.DMA,
            pltpu.SemaphoreType.DMA,
            pltpu.SemaphoreType.DMA,
            pltpu.SemaphoreType.DMA,
            pltpu.SemaphoreType.DMA,
            pltpu.SemaphoreType.DMA,
            pltpu.SemaphoreType.DMA,
            pltpu.SemaphoreType.DMA,
            pltpu.SemaphoreType.DMA,
        ],
    )
    def body(src_hbm, dst_hbm, a_hbm, b_hbm, c_hbm, s_out,
             is0, id0, ra0, rb0, rc0,
             is1, id1, ra1, rb1, rc1,
             is2, id2, ra2, rb2, rc2, s_sh,
             semA0, semB0, semC0, semA1, semB1, semC1, semA2, semB2, semC2):
        c = lax.axis_index("c")
        s = lax.axis_index("s")
        z16 = jnp.zeros((16,), jnp.float32)
        sets = [
            (is0, id0, ra0, rb0, rc0, semA0, semB0, semC0),
            (is1, id1, ra1, rb1, rc1, semA1, semB1, semC1),
            (is2, id2, ra2, rb2, rc2, semA2, semB2, semC2),
        ]

        def zero_row(i, _):
            for j in range(H // 16):
                ra0[i, pl.ds(j * 16, 16)] = z16
            return 0

        lax.fori_loop(0, K, zero_row, 0)

        nb = s * RPT
        nfull = RPT // K

        def zero_copy(q, _):
            pltpu.sync_copy(ra0, s_sh.at[pl.ds(nb + q * K, K)])
            return 0

        lax.fori_loop(0, nfull, zero_copy, 0)
        rem = RPT - nfull * K
        if rem:
            pltpu.sync_copy(ra0.at[pl.ds(0, rem)], s_sh.at[pl.ds(nb + nfull * K, rem)])
        if TAIL:
            @pl.when(s == NS - 1)
            def _zero_tail():
                pltpu.sync_copy(ra0.at[pl.ds(0, TAIL)], s_sh.at[pl.ds(NS * RPT, TAIL)])

        plsc.subcore_barrier()

        base_e = (c * NS + s) * EW

        def start_gathers(b, g):
            isb, idb, rab, rbb, rcb, sA, sB, sC = sets[b]
            eb = base_e + g * K
            pltpu.sync_copy(src_hbm.at[pl.ds(eb, K)], isb)
            pltpu.sync_copy(dst_hbm.at[pl.ds(eb, K)], idb)
            pltpu.async_copy(a_hbm.at[isb], rab, sA)
            pltpu.async_copy(b_hbm.at[idb], rbb, sB)
            pltpu.async_copy(c_hbm.at[pl.ds(eb, K)], rcb, sC)

        def finish_chunk(b, g):
            isb, idb, rab, rbb, rcb, sA, sB, sC = sets[b]
            pltpu.make_async_copy(a_hbm.at[isb], rab, sA).wait()
            pltpu.make_async_copy(b_hbm.at[idb], rbb, sB).wait()
            pltpu.make_async_copy(c_hbm.at[pl.ds(0, K)], rcb, sC).wait()

            @plsc.parallel_loop(0, K, unroll=2)
            def row(i):
                for j in range(H // 16):
                    sl = pl.ds(j * 16, 16)
                    rab[i, sl] = jnp.maximum(rab[i, sl] + rbb[i, sl] + rcb[i, sl], 0.0)

            pltpu.sync_copy(rab, s_sh.at[idb], add=True)

            @pl.when(g + 3 < NCHUNK)
            def _prefetch():
                start_gathers(b, g + 3)

        start_gathers(0, 0)
        start_gathers(1, 1)
        start_gathers(2, 2)

        NT = NCHUNK // 3

        def triple(t, _):
            finish_chunk(0, 3 * t)
            finish_chunk(1, 3 * t + 1)
            finish_chunk(2, 3 * t + 2)
            return 0

        lax.fori_loop(0, NT, triple, 0)
        for g_epi in range(3 * NT, NCHUNK):
            finish_chunk(g_epi % 3, g_epi)

        plsc.subcore_barrier()
        pltpu.sync_copy(s_sh.at[pl.ds(nb, RPT)], s_out.at[c, pl.ds(nb, RPT)])
        if TAIL:
            @pl.when(s == NS - 1)
            def _out_tail():
                pltpu.sync_copy(s_sh.at[pl.ds(NS * RPT, TAIL)],
                                s_out.at[c, pl.ds(NS * RPT, TAIL)])

    return body(src, dst, A, B, C)[0]


def _edge_deg(dst, N, H):
    E = dst.shape[0]
    KD = 80
    EW = E // (NC * NS)
    NCHUNK = EW // KD
    RPT = (N // NS) // 8 * 8
    TAIL = N - NS * RPT
    mesh = plsc.VectorSubcoreMesh(core_axis_name="c", subcore_axis_name="s")

    @functools.partial(
        pl.kernel,
        out_type=[jax.ShapeDtypeStruct((NC, N, H), jnp.float32)],
        mesh=mesh,
        scratch_types=[
            pltpu.VMEM((KD,), jnp.int32),
            pltpu.VMEM((KD,), jnp.int32),
            pltpu.VMEM((KD, H), jnp.float32),
            pltpu.VMEM_SHARED((N, H), jnp.float32),
            pltpu.SemaphoreType.DMA,
            pltpu.SemaphoreType.DMA,
        ],
    )
    def body(dst_hbm, deg_out, id0, id1, dval, d_sh, semS0, semS1):
        c = lax.axis_index("c")
        s = lax.axis_index("s")
        z16 = jnp.zeros((16,), jnp.float32)

        def zero_row(i, _):
            for j in range(H // 16):
                dval[i, pl.ds(j * 16, 16)] = z16
            return 0

        lax.fori_loop(0, KD, zero_row, 0)

        nb = s * RPT
        nfull = RPT // KD

        def zero_copy(q, _):
            pltpu.sync_copy(dval, d_sh.at[pl.ds(nb + q * KD, KD)])
            return 0

        lax.fori_loop(0, nfull, zero_copy, 0)
        rem = RPT - nfull * KD
        if rem:
            pltpu.sync_copy(dval.at[pl.ds(0, rem)], d_sh.at[pl.ds(nb + nfull * KD, rem)])
        if TAIL:
            @pl.when(s == NS - 1)
            def _zero_tail():
                pltpu.sync_copy(dval.at[pl.ds(0, TAIL)], d_sh.at[pl.ds(NS * RPT, TAIL)])

        e0 = jnp.where(lax.iota(jnp.int32, 16) == 0, 1.0, 0.0).astype(jnp.float32)

        def ones_row(i, _):
            dval[i, pl.ds(0, 16)] = e0
            return 0

        lax.fori_loop(0, KD, ones_row, 0)

        plsc.subcore_barrier()

        base_e = (c * NS + s) * EW
        sets = [(id0, semS0), (id1, semS1)]

        def half(b, gp, g):
            idb, sS = sets[b]

            @pl.when(gp > 0)
            def _drain():
                pltpu.make_async_copy(dval, d_sh.at[idb], sS).wait()

            pltpu.sync_copy(dst_hbm.at[pl.ds(base_e + g * KD, KD)], idb)
            pltpu.async_copy(dval, d_sh.at[idb], sS, add=True)

        def pair(gp, _):
            half(0, gp, 2 * gp)
            half(1, gp, 2 * gp + 1)
            return 0

        NPAIR = NCHUNK // 2
        lax.fori_loop(0, NPAIR, pair, 0)
        if NCHUNK % 2:
            half(0, NPAIR, NCHUNK - 1)
        pltpu.make_async_copy(dval, d_sh.at[id0], semS0).wait()
        if NCHUNK > 1:
            pltpu.make_async_copy(dval, d_sh.at[id1], semS1).wait()

        plsc.subcore_barrier()
        pltpu.sync_copy(d_sh.at[pl.ds(nb, RPT)], deg_out.at[c, pl.ds(nb, RPT)])
        if TAIL:
            @pl.when(s == NS - 1)
            def _out_tail():
                pltpu.sync_copy(d_sh.at[pl.ds(NS * RPT, TAIL)],
                                deg_out.at[c, pl.ds(NS * RPT, TAIL)])

    return body(dst)[0]


def _node_post(h, S, deg, W2m, b2m, W1uh, W1ua, b1u, W2u, b2u, ln_g, ln_b):
    N, D = h.shape
    H = W2m.shape[0]
    BN = 2000

    def body(h_ref, s_ref, d_ref, w2m_ref, b2m_ref, w1uh_ref, w1ua_ref,
             b1u_ref, w2u_ref, b2u_ref, lng_ref, lnb_ref, out_ref):
        hb = h_ref[...]
        Sb = s_ref[0] + s_ref[1]
        degb = d_ref[0, :, 0:1] + d_ref[1, :, 0:1]
        agg = (
            jnp.dot(Sb, w2m_ref[...], preferred_element_type=jnp.float32)
            + degb * b2m_ref[...]
        )
        y = (
            jnp.dot(hb, w1uh_ref[...], preferred_element_type=jnp.float32)
            + jnp.dot(agg, w1ua_ref[...], preferred_element_type=jnp.float32)
            + b1u_ref[...]
        )
        y = jnp.maximum(y, 0.0)
        hn = jnp.dot(y, w2u_ref[...], preferred_element_type=jnp.float32) + b2u_ref[...]
        z = hb + hn
        mean = jnp.mean(z, axis=1, keepdims=True)
        zc = z - mean
        var = jnp.mean(zc * zc, axis=1, keepdims=True)
        out_ref[...] = zc * lax.rsqrt(var + EPS) * lng_ref[...] + lnb_ref[...]

    full = lambda r, c: pl.BlockSpec((r, c), lambda i: (0, 0))
    return pl.pallas_call(
        body,
        grid=(N // BN,),
        in_specs=[
            pl.BlockSpec((BN, D), lambda i: (i, 0)),
            pl.BlockSpec((NC, BN, H), lambda i: (0, i, 0)),
            pl.BlockSpec((NC, BN, H), lambda i: (0, i, 0)),
            full(H, D), full(1, D), full(D, H), full(H, H), full(1, H),
            full(H, D), full(1, D), full(1, D), full(1, D),
        ],
        out_specs=pl.BlockSpec((BN, D), lambda i: (i, 0)),
        out_shape=jax.ShapeDtypeStruct((N, D), jnp.float32),
    )(h, S, deg, W2m, b2m.reshape(1, -1), W1uh, W1ua, b1u.reshape(1, -1),
      W2u, b2u.reshape(1, -1), ln_g.reshape(1, -1), ln_b.reshape(1, -1))


def kernel(h, edge_index, edge_attr, num_nodes, W1m, b1m, g1m, be1m, W2m, b2m,
           W1u, b1u, g1u, be1u, W2u, b2u, ln_g, ln_b):
    N, D = h.shape
    H = W1m.shape[1]

    s1m = g1m * (1.0 / jnp.sqrt(1.0 + EPS))
    Wsrc = W1m[:D] * s1m
    Wdst = W1m[D:2 * D] * s1m
    We = W1m[2 * D:] * s1m
    b1m_f = b1m * s1m + be1m
    s1u = g1u * (1.0 / jnp.sqrt(1.0 + EPS))
    W1uh = W1u[:D] * s1u
    W1ua = W1u[D:] * s1u
    b1u_f = b1u * s1u + be1u

    src = edge_index[0]
    dst = edge_index[1]

    A, B = _node_pre(h, Wsrc, Wdst)
    C = _edge_pre(edge_attr, We, b1m_f)
    S = _edge_msg(src, dst, A, B, C)
    deg = _edge_deg(dst, N, H)
    return _node_post(h, S, deg, W2m, b2m, W1uh, W1ua, b1u_f, W2u, b2u, ln_g, ln_b)


# speedup vs baseline: 1.1985x; 1.1248x over previous
import functools

import jax
import jax.numpy as jnp
from jax import lax
from jax.experimental import pallas as pl
from jax.experimental.pallas import tpu as pltpu
from jax.experimental.pallas import tpu_sc as plsc

EPS = 1e-5

NC = 2
NS = 16
K = 64


def _node_pre(h, Ws, Wd):
    N, D = h.shape
    H = Ws.shape[1]
    BN = 2000

    def body(h_ref, ws_ref, wd_ref, a_ref, b_ref):
        hb = h_ref[...]
        a_ref[...] = jnp.dot(hb, ws_ref[...], preferred_element_type=jnp.float32)
        b_ref[...] = jnp.dot(hb, wd_ref[...], preferred_element_type=jnp.float32)

    return pl.pallas_call(
        body,
        grid=(N // BN,),
        in_specs=[
            pl.BlockSpec((BN, D), lambda i: (i, 0)),
            pl.BlockSpec((D, H), lambda i: (0, 0)),
            pl.BlockSpec((D, H), lambda i: (0, 0)),
        ],
        out_specs=[
            pl.BlockSpec((BN, H), lambda i: (i, 0)),
            pl.BlockSpec((BN, H), lambda i: (i, 0)),
        ],
        out_shape=[jax.ShapeDtypeStruct((N, H), jnp.float32)] * 2,
    )(h, Ws, Wd)


def _edge_pre(ea, We, b):
    E, ED = ea.shape
    H = We.shape[1]
    BE = 8000

    def body(ea_ref, we_ref, b_ref, c_ref):
        c_ref[...] = (
            jnp.dot(ea_ref[...], we_ref[...], preferred_element_type=jnp.float32)
            + b_ref[...]
        )

    return pl.pallas_call(
        body,
        grid=(E // BE,),
        in_specs=[
            pl.BlockSpec((BE, ED), lambda i: (i, 0)),
            pl.BlockSpec((ED, H), lambda i: (0, 0)),
            pl.BlockSpec((1, H), lambda i: (0, 0)),
        ],
        out_specs=pl.BlockSpec((BE, H), lambda i: (i, 0)),
        out_shape=jax.ShapeDtypeStruct((E, H), jnp.float32),
    )(ea, We, b.reshape(1, H))


def _edge_msg(src, dst, A, B, C):
    N, H = A.shape
    E = src.shape[0]
    TCH = E // K
    CW = TCH // (NC * NS)
    XTRA = TCH % (NC * NS)
    RPT = (N // NS) // 8 * 8
    TAIL = N - NS * RPT
    mesh = plsc.VectorSubcoreMesh(core_axis_name="c", subcore_axis_name="s")

    @functools.partial(
        pl.kernel,
        out_type=[jax.ShapeDtypeStruct((NC, N, H), jnp.float32)],
        mesh=mesh,
        scratch_types=[
            pltpu.VMEM((K,), jnp.int32),
            pltpu.VMEM((K,), jnp.int32),
            pltpu.VMEM((K, H), jnp.float32),
            pltpu.VMEM((K, H), jnp.float32),
            pltpu.VMEM((K, H), jnp.float32),
            pltpu.VMEM((K,), jnp.int32),
            pltpu.VMEM((K,), jnp.int32),
            pltpu.VMEM((K, H), jnp.float32),
            pltpu.VMEM((K, H), jnp.float32),
            pltpu.VMEM((K, H), jnp.float32),
            pltpu.VMEM_SHARED((N, H), jnp.float32),
            pltpu.SemaphoreType.DMA,
            pltpu.SemaphoreType.DMA,
            pltpu.SemaphoreType.DMA,
            pltpu.SemaphoreType.DMA,
            pltpu.SemaphoreType.DMA,
            pltpu.SemaphoreType.DMA,
        ],
    )
    def body(src_hbm, dst_hbm, a_hbm, b_hbm, c_hbm, s_out,
             is0, id0, ra0, rb0, rc0,
             is1, id1, ra1, rb1, rc1, s_sh,
             semA0, semB0, semC0, semA1, semB1, semC1):
        c = lax.axis_index("c")
        s = lax.axis_index("s")
        z16 = jnp.zeros((16,), jnp.float32)
        sets = [
            (is0, id0, ra0, rb0, rc0, semA0, semB0, semC0),
            (is1, id1, ra1, rb1, rc1, semA1, semB1, semC1),
        ]

        def zero_row(i, _):
            for j in range(H // 16):
                ra0[i, pl.ds(j * 16, 16)] = z16
            return 0

        lax.fori_loop(0, K, zero_row, 0)

        nb = s * RPT
        nfull = RPT // K

        def zero_copy(q, _):
            pltpu.sync_copy(ra0, s_sh.at[pl.ds(nb + q * K, K)])
            return 0

        lax.fori_loop(0, nfull, zero_copy, 0)
        rem = RPT - nfull * K
        if rem:
            pltpu.sync_copy(ra0.at[pl.ds(0, rem)], s_sh.at[pl.ds(nb + nfull * K, rem)])
        if TAIL:
            @pl.when(s == NS - 1)
            def _zero_tail():
                pltpu.sync_copy(ra0.at[pl.ds(0, TAIL)], s_sh.at[pl.ds(NS * RPT, TAIL)])

        plsc.subcore_barrier()

        w = c * NS + s
        my_start = w * CW + jnp.minimum(w, XTRA)
        my_n = CW + (w < XTRA).astype(jnp.int32)

        def start_gathers(b, g):
            isb, idb, rab, rbb, rcb, sA, sB, sC = sets[b]
            eb = (my_start + g) * K
            pltpu.sync_copy(src_hbm.at[pl.ds(eb, K)], isb)
            pltpu.sync_copy(dst_hbm.at[pl.ds(eb, K)], idb)
            pltpu.async_copy(a_hbm.at[isb], rab, sA)
            pltpu.async_copy(b_hbm.at[idb], rbb, sB)
            pltpu.async_copy(c_hbm.at[pl.ds(eb, K)], rcb, sC)

        def finish_chunk(b, g):
            isb, idb, rab, rbb, rcb, sA, sB, sC = sets[b]
            pltpu.make_async_copy(a_hbm.at[isb], rab, sA).wait()
            pltpu.make_async_copy(b_hbm.at[idb], rbb, sB).wait()
            pltpu.make_async_copy(c_hbm.at[pl.ds(0, K)], rcb, sC).wait()

            @plsc.parallel_loop(0, K, unroll=2)
            def row(i):
                for j in range(H // 16):
                    sl = pl.ds(j * 16, 16)
                    rab[i, sl] = jnp.maximum(rab[i, sl] + rbb[i, sl] + rcb[i, sl], 0.0)

            pltpu.sync_copy(rab, s_sh.at[idb], add=True)

            @pl.when(g + 2 < my_n)
            def _prefetch():
                start_gathers(b, g + 2)

        start_gathers(0, 0)
        start_gathers(1, 1)

        def pair(gp, _):
            finish_chunk(0, 2 * gp)
            finish_chunk(1, 2 * gp + 1)
            return 0

        lax.fori_loop(0, my_n // 2, pair, 0)

        @pl.when(my_n % 2 == 1)
        def _odd_tail():
            finish_chunk(0, my_n - 1)

        plsc.subcore_barrier()
        pltpu.sync_copy(s_sh.at[pl.ds(nb, RPT)], s_out.at[c, pl.ds(nb, RPT)])
        if TAIL:
            @pl.when(s == NS - 1)
            def _out_tail():
                pltpu.sync_copy(s_sh.at[pl.ds(NS * RPT, TAIL)],
                                s_out.at[c, pl.ds(NS * RPT, TAIL)])

    return body(src, dst, A, B, C)[0]


def _edge_deg(dst, N, H):
    E = dst.shape[0]
    KD = 80
    EW = E // (NC * NS)
    NCHUNK = EW // KD
    RPT = (N // NS) // 8 * 8
    TAIL = N - NS * RPT
    mesh = plsc.VectorSubcoreMesh(core_axis_name="c", subcore_axis_name="s")

    @functools.partial(
        pl.kernel,
        out_type=[jax.ShapeDtypeStruct((NC, N, H), jnp.float32)],
        mesh=mesh,
        scratch_types=[
            pltpu.VMEM((KD,), jnp.int32),
            pltpu.VMEM((KD,), jnp.int32),
            pltpu.VMEM((KD, H), jnp.float32),
            pltpu.VMEM_SHARED((N, H), jnp.float32),
            pltpu.SemaphoreType.DMA,
            pltpu.SemaphoreType.DMA,
        ],
    )
    def body(dst_hbm, deg_out, id0, id1, dval, d_sh, semS0, semS1):
        c = lax.axis_index("c")
        s = lax.axis_index("s")
        z16 = jnp.zeros((16,), jnp.float32)

        def zero_row(i, _):
            for j in range(H // 16):
                dval[i, pl.ds(j * 16, 16)] = z16
            return 0

        lax.fori_loop(0, KD, zero_row, 0)

        nb = s * RPT
        nfull = RPT // KD

        def zero_copy(q, _):
            pltpu.sync_copy(dval, d_sh.at[pl.ds(nb + q * KD, KD)])
            return 0

        lax.fori_loop(0, nfull, zero_copy, 0)
        rem = RPT - nfull * KD
        if rem:
            pltpu.sync_copy(dval.at[pl.ds(0, rem)], d_sh.at[pl.ds(nb + nfull * KD, rem)])
        if TAIL:
            @pl.when(s == NS - 1)
            def _zero_tail():
                pltpu.sync_copy(dval.at[pl.ds(0, TAIL)], d_sh.at[pl.ds(NS * RPT, TAIL)])

        e0 = jnp.where(lax.iota(jnp.int32, 16) == 0, 1.0, 0.0).astype(jnp.float32)

        def ones_row(i, _):
            dval[i, pl.ds(0, 16)] = e0
            return 0

        lax.fori_loop(0, KD, ones_row, 0)

        plsc.subcore_barrier()

        base_e = (c * NS + s) * EW
        sets = [(id0, semS0), (id1, semS1)]

        def half(b, gp, g):
            idb, sS = sets[b]

            @pl.when(gp > 0)
            def _drain():
                pltpu.make_async_copy(dval, d_sh.at[idb], sS).wait()

            pltpu.sync_copy(dst_hbm.at[pl.ds(base_e + g * KD, KD)], idb)
            pltpu.async_copy(dval, d_sh.at[idb], sS, add=True)

        def pair(gp, _):
            half(0, gp, 2 * gp)
            half(1, gp, 2 * gp + 1)
            return 0

        NPAIR = NCHUNK // 2
        lax.fori_loop(0, NPAIR, pair, 0)
        if NCHUNK % 2:
            half(0, NPAIR, NCHUNK - 1)
        pltpu.make_async_copy(dval, d_sh.at[id0], semS0).wait()
        if NCHUNK > 1:
            pltpu.make_async_copy(dval, d_sh.at[id1], semS1).wait()

        plsc.subcore_barrier()
        pltpu.sync_copy(d_sh.at[pl.ds(nb, RPT)], deg_out.at[c, pl.ds(nb, RPT)])
        if TAIL:
            @pl.when(s == NS - 1)
            def _out_tail():
                pltpu.sync_copy(d_sh.at[pl.ds(NS * RPT, TAIL)],
                                deg_out.at[c, pl.ds(NS * RPT, TAIL)])

    return body(dst)[0]


def _node_post(h, S, deg, W2m, b2m, W1uh, W1ua, b1u, W2u, b2u, ln_g, ln_b):
    N, D = h.shape
    H = W2m.shape[0]
    BN = 2000

    def body(h_ref, s_ref, d_ref, w2m_ref, b2m_ref, w1uh_ref, w1ua_ref,
             b1u_ref, w2u_ref, b2u_ref, lng_ref, lnb_ref, out_ref):
        hb = h_ref[...]
        Sb = s_ref[0] + s_ref[1]
        degb = d_ref[0, :, 0:1] + d_ref[1, :, 0:1]
        agg = (
            jnp.dot(Sb, w2m_ref[...], preferred_element_type=jnp.float32)
            + degb * b2m_ref[...]
        )
        y = (
            jnp.dot(hb, w1uh_ref[...], preferred_element_type=jnp.float32)
            + jnp.dot(agg, w1ua_ref[...], preferred_element_type=jnp.float32)
            + b1u_ref[...]
        )
        y = jnp.maximum(y, 0.0)
        hn = jnp.dot(y, w2u_ref[...], preferred_element_type=jnp.float32) + b2u_ref[...]
        z = hb + hn
        mean = jnp.mean(z, axis=1, keepdims=True)
        zc = z - mean
        var = jnp.mean(zc * zc, axis=1, keepdims=True)
        out_ref[...] = zc * lax.rsqrt(var + EPS) * lng_ref[...] + lnb_ref[...]

    full = lambda r, c: pl.BlockSpec((r, c), lambda i: (0, 0))
    return pl.pallas_call(
        body,
        grid=(N // BN,),
        in_specs=[
            pl.BlockSpec((BN, D), lambda i: (i, 0)),
            pl.BlockSpec((NC, BN, H), lambda i: (0, i, 0)),
            pl.BlockSpec((NC, BN, H), lambda i: (0, i, 0)),
            full(H, D), full(1, D), full(D, H), full(H, H), full(1, H),
            full(H, D), full(1, D), full(1, D), full(1, D),
        ],
        out_specs=pl.BlockSpec((BN, D), lambda i: (i, 0)),
        out_shape=jax.ShapeDtypeStruct((N, D), jnp.float32),
    )(h, S, deg, W2m, b2m.reshape(1, -1), W1uh, W1ua, b1u.reshape(1, -1),
      W2u, b2u.reshape(1, -1), ln_g.reshape(1, -1), ln_b.reshape(1, -1))


def kernel(h, edge_index, edge_attr, num_nodes, W1m, b1m, g1m, be1m, W2m, b2m,
           W1u, b1u, g1u, be1u, W2u, b2u, ln_g, ln_b):
    N, D = h.shape
    H = W1m.shape[1]

    s1m = g1m * (1.0 / jnp.sqrt(1.0 + EPS))
    Wsrc = W1m[:D] * s1m
    Wdst = W1m[D:2 * D] * s1m
    We = W1m[2 * D:] * s1m
    b1m_f = b1m * s1m + be1m
    s1u = g1u * (1.0 / jnp.sqrt(1.0 + EPS))
    W1uh = W1u[:D] * s1u
    W1ua = W1u[D:] * s1u
    b1u_f = b1u * s1u + be1u

    src = edge_index[0]
    dst = edge_index[1]

    A, B = _node_pre(h, Wsrc, Wdst)
    C = _edge_pre(edge_attr, We, b1m_f)
    S = _edge_msg(src, dst, A, B, C)
    deg = _edge_deg(dst, N, H)
    return _node_post(h, S, deg, W2m, b2m, W1uh, W1ua, b1u_f, W2u, b2u, ln_g, ln_b)


# speedup vs baseline: 1.2086x; 1.0084x over previous
import functools

import jax
import jax.numpy as jnp
from jax import lax
from jax.experimental import pallas as pl
from jax.experimental.pallas import tpu as pltpu
from jax.experimental.pallas import tpu_sc as plsc

EPS = 1e-5

NC = 2
NS = 16
K = 64


def _pre_all(h, Ws, Wd, ea, We, b):
    N, D = h.shape
    E, ED = ea.shape
    H = We.shape[1]
    BE = 8000
    BN = 2000
    NBA = N // BN
    clamp = lambda i: (jnp.minimum(i, NBA - 1), 0)

    def body(ea_ref, we_ref, b_ref, h_ref, ws_ref, wd_ref, c_ref, a_ref, b2_ref):
        c_ref[...] = (
            jnp.dot(ea_ref[...], we_ref[...], preferred_element_type=jnp.float32)
            + b_ref[...]
        )
        hb = h_ref[...]
        a_ref[...] = jnp.dot(hb, ws_ref[...], preferred_element_type=jnp.float32)
        b2_ref[...] = jnp.dot(hb, wd_ref[...], preferred_element_type=jnp.float32)

    return pl.pallas_call(
        body,
        grid=(E // BE,),
        in_specs=[
            pl.BlockSpec((BE, ED), lambda i: (i, 0)),
            pl.BlockSpec((ED, H), lambda i: (0, 0)),
            pl.BlockSpec((1, H), lambda i: (0, 0)),
            pl.BlockSpec((BN, D), clamp),
            pl.BlockSpec((D, H), lambda i: (0, 0)),
            pl.BlockSpec((D, H), lambda i: (0, 0)),
        ],
        out_specs=[
            pl.BlockSpec((BE, H), lambda i: (i, 0)),
            pl.BlockSpec((BN, H), clamp),
            pl.BlockSpec((BN, H), clamp),
        ],
        out_shape=[
            jax.ShapeDtypeStruct((E, H), jnp.float32),
            jax.ShapeDtypeStruct((N, H), jnp.float32),
            jax.ShapeDtypeStruct((N, H), jnp.float32),
        ],
    )(ea, We, b.reshape(1, H), h, Ws, Wd)


def _edge_msg(src, dst, A, B, C):
    N, H = A.shape
    E = src.shape[0]
    TCH = E // K
    CW = TCH // (NC * NS)
    XTRA = TCH % (NC * NS)
    RPT = (N // NS) // 8 * 8
    TAIL = N - NS * RPT
    mesh = plsc.VectorSubcoreMesh(core_axis_name="c", subcore_axis_name="s")

    @functools.partial(
        pl.kernel,
        out_type=[jax.ShapeDtypeStruct((NC, N, H), jnp.float32)],
        mesh=mesh,
        scratch_types=[
            pltpu.VMEM((K,), jnp.int32),
            pltpu.VMEM((K,), jnp.int32),
            pltpu.VMEM((K, H), jnp.float32),
            pltpu.VMEM((K, H), jnp.float32),
            pltpu.VMEM((K, H), jnp.float32),
            pltpu.VMEM((K,), jnp.int32),
            pltpu.VMEM((K,), jnp.int32),
            pltpu.VMEM((K, H), jnp.float32),
            pltpu.VMEM((K, H), jnp.float32),
            pltpu.VMEM((K, H), jnp.float32),
            pltpu.VMEM_SHARED((N, H), jnp.float32),
            pltpu.SemaphoreType.DMA,
            pltpu.SemaphoreType.DMA,
            pltpu.SemaphoreType.DMA,
            pltpu.SemaphoreType.DMA,
            pltpu.SemaphoreType.DMA,
            pltpu.SemaphoreType.DMA,
        ],
    )
    def body(src_hbm, dst_hbm, a_hbm, b_hbm, c_hbm, s_out,
             is0, id0, ra0, rb0, rc0,
             is1, id1, ra1, rb1, rc1, s_sh,
             semA0, semB0, semC0, semA1, semB1, semC1):
        c = lax.axis_index("c")
        s = lax.axis_index("s")
        z16 = jnp.zeros((16,), jnp.float32)
        sets = [
            (is0, id0, ra0, rb0, rc0, semA0, semB0, semC0),
            (is1, id1, ra1, rb1, rc1, semA1, semB1, semC1),
        ]

        def zero_row(i, _):
            for j in range(H // 16):
                ra0[i, pl.ds(j * 16, 16)] = z16
            return 0

        lax.fori_loop(0, K, zero_row, 0)

        nb = s * RPT
        nfull = RPT // K

        def zero_copy(q, _):
            pltpu.sync_copy(ra0, s_sh.at[pl.ds(nb + q * K, K)])
            return 0

        lax.fori_loop(0, nfull, zero_copy, 0)
        rem = RPT - nfull * K
        if rem:
            pltpu.sync_copy(ra0.at[pl.ds(0, rem)], s_sh.at[pl.ds(nb + nfull * K, rem)])
        if TAIL:
            @pl.when(s == NS - 1)
            def _zero_tail():
                pltpu.sync_copy(ra0.at[pl.ds(0, TAIL)], s_sh.at[pl.ds(NS * RPT, TAIL)])

        plsc.subcore_barrier()

        w = c * NS + s
        my_start = w * CW + jnp.minimum(w, XTRA)
        my_n = CW + (w < XTRA).astype(jnp.int32)

        def start_gathers(b, g):
            isb, idb, rab, rbb, rcb, sA, sB, sC = sets[b]
            eb = (my_start + g) * K
            pltpu.sync_copy(src_hbm.at[pl.ds(eb, K)], isb)
            pltpu.sync_copy(dst_hbm.at[pl.ds(eb, K)], idb)
            pltpu.async_copy(a_hbm.at[isb], rab, sA)
            pltpu.async_copy(b_hbm.at[idb], rbb, sB)
            pltpu.async_copy(c_hbm.at[pl.ds(eb, K)], rcb, sC)

        def finish_chunk(b, g):
            isb, idb, rab, rbb, rcb, sA, sB, sC = sets[b]
            pltpu.make_async_copy(a_hbm.at[isb], rab, sA).wait()
            pltpu.make_async_copy(b_hbm.at[idb], rbb, sB).wait()
            pltpu.make_async_copy(c_hbm.at[pl.ds(0, K)], rcb, sC).wait()

            @plsc.parallel_loop(0, K, unroll=4)
            def row(i):
                for j in range(H // 16):
                    sl = pl.ds(j * 16, 16)
                    rab[i, sl] = jnp.maximum(rab[i, sl] + rbb[i, sl] + rcb[i, sl], 0.0)

            pltpu.sync_copy(rab, s_sh.at[idb], add=True)

            @pl.when(g + 2 < my_n)
            def _prefetch():
                start_gathers(b, g + 2)

        start_gathers(0, 0)
        start_gathers(1, 1)

        def pair(gp, _):
            finish_chunk(0, 2 * gp)
            finish_chunk(1, 2 * gp + 1)
            return 0

        lax.fori_loop(0, my_n // 2, pair, 0)

        @pl.when(my_n % 2 == 1)
        def _odd_tail():
            finish_chunk(0, my_n - 1)

        plsc.subcore_barrier()
        pltpu.sync_copy(s_sh.at[pl.ds(nb, RPT)], s_out.at[c, pl.ds(nb, RPT)])
        if TAIL:
            @pl.when(s == NS - 1)
            def _out_tail():
                pltpu.sync_copy(s_sh.at[pl.ds(NS * RPT, TAIL)],
                                s_out.at[c, pl.ds(NS * RPT, TAIL)])

    return body(src, dst, A, B, C)[0]


def _edge_deg(dst, N, H):
    E = dst.shape[0]
    KD = 80
    EW = E // (NC * NS)
    NCHUNK = EW // KD
    RPT = (N // NS) // 8 * 8
    TAIL = N - NS * RPT
    mesh = plsc.VectorSubcoreMesh(core_axis_name="c", subcore_axis_name="s")

    @functools.partial(
        pl.kernel,
        out_type=[jax.ShapeDtypeStruct((NC, N, H), jnp.float32)],
        mesh=mesh,
        scratch_types=[
            pltpu.VMEM((KD,), jnp.int32),
            pltpu.VMEM((KD,), jnp.int32),
            pltpu.VMEM((KD, H), jnp.float32),
            pltpu.VMEM_SHARED((N, H), jnp.float32),
            pltpu.SemaphoreType.DMA,
            pltpu.SemaphoreType.DMA,
        ],
    )
    def body(dst_hbm, deg_out, id0, id1, dval, d_sh, semS0, semS1):
        c = lax.axis_index("c")
        s = lax.axis_index("s")
        z16 = jnp.zeros((16,), jnp.float32)

        def zero_row(i, _):
            for j in range(H // 16):
                dval[i, pl.ds(j * 16, 16)] = z16
            return 0

        lax.fori_loop(0, KD, zero_row, 0)

        nb = s * RPT
        nfull = RPT // KD

        def zero_copy(q, _):
            pltpu.sync_copy(dval, d_sh.at[pl.ds(nb + q * KD, KD)])
            return 0

        lax.fori_loop(0, nfull, zero_copy, 0)
        rem = RPT - nfull * KD
        if rem:
            pltpu.sync_copy(dval.at[pl.ds(0, rem)], d_sh.at[pl.ds(nb + nfull * KD, rem)])
        if TAIL:
            @pl.when(s == NS - 1)
            def _zero_tail():
                pltpu.sync_copy(dval.at[pl.ds(0, TAIL)], d_sh.at[pl.ds(NS * RPT, TAIL)])

        e0 = jnp.where(lax.iota(jnp.int32, 16) == 0, 1.0, 0.0).astype(jnp.float32)

        def ones_row(i, _):
            dval[i, pl.ds(0, 16)] = e0
            return 0

        lax.fori_loop(0, KD, ones_row, 0)

        plsc.subcore_barrier()

        base_e = (c * NS + s) * EW
        sets = [(id0, semS0), (id1, semS1)]

        def half(b, gp, g):
            idb, sS = sets[b]

            @pl.when(gp > 0)
            def _drain():
                pltpu.make_async_copy(dval, d_sh.at[idb], sS).wait()

            pltpu.sync_copy(dst_hbm.at[pl.ds(base_e + g * KD, KD)], idb)
            pltpu.async_copy(dval, d_sh.at[idb], sS, add=True)

        def pair(gp, _):
            half(0, gp, 2 * gp)
            half(1, gp, 2 * gp + 1)
            return 0

        NPAIR = NCHUNK // 2
        lax.fori_loop(0, NPAIR, pair, 0)
        if NCHUNK % 2:
            half(0, NPAIR, NCHUNK - 1)
        pltpu.make_async_copy(dval, d_sh.at[id0], semS0).wait()
        if NCHUNK > 1:
            pltpu.make_async_copy(dval, d_sh.at[id1], semS1).wait()

        plsc.subcore_barrier()
        pltpu.sync_copy(d_sh.at[pl.ds(nb, RPT)], deg_out.at[c, pl.ds(nb, RPT)])
        if TAIL:
            @pl.when(s == NS - 1)
            def _out_tail():
                pltpu.sync_copy(d_sh.at[pl.ds(NS * RPT, TAIL)],
                                deg_out.at[c, pl.ds(NS * RPT, TAIL)])

    return body(dst)[0]


def _node_post(h, S, deg, W2m, b2m, W1uh, W1ua, b1u, W2u, b2u, ln_g, ln_b):
    N, D = h.shape
    H = W2m.shape[0]
    BN = 2000

    def body(h_ref, s_ref, d_ref, w2m_ref, b2m_ref, w1uh_ref, w1ua_ref,
             b1u_ref, w2u_ref, b2u_ref, lng_ref, lnb_ref, out_ref):
        hb = h_ref[...]
        Sb = s_ref[0] + s_ref[1]
        degb = d_ref[0, :, 0:1] + d_ref[1, :, 0:1]
        agg = (
            jnp.dot(Sb, w2m_ref[...], preferred_element_type=jnp.float32)
            + degb * b2m_ref[...]
        )
        y = (
            jnp.dot(hb, w1uh_ref[...], preferred_element_type=jnp.float32)
            + jnp.dot(agg, w1ua_ref[...], preferred_element_type=jnp.float32)
            + b1u_ref[...]
        )
        y = jnp.maximum(y, 0.0)
        hn = jnp.dot(y, w2u_ref[...], preferred_element_type=jnp.float32) + b2u_ref[...]
        z = hb + hn
        mean = jnp.mean(z, axis=1, keepdims=True)
        zc = z - mean
        var = jnp.mean(zc * zc, axis=1, keepdims=True)
        out_ref[...] = zc * lax.rsqrt(var + EPS) * lng_ref[...] + lnb_ref[...]

    full = lambda r, c: pl.BlockSpec((r, c), lambda i: (0, 0))
    return pl.pallas_call(
        body,
        grid=(N // BN,),
        in_specs=[
            pl.BlockSpec((BN, D), lambda i: (i, 0)),
            pl.BlockSpec((NC, BN, H), lambda i: (0, i, 0)),
            pl.BlockSpec((NC, BN, H), lambda i: (0, i, 0)),
            full(H, D), full(1, D), full(D, H), full(H, H), full(1, H),
            full(H, D), full(1, D), full(1, D), full(1, D),
        ],
        out_specs=pl.BlockSpec((BN, D), lambda i: (i, 0)),
        out_shape=jax.ShapeDtypeStruct((N, D), jnp.float32),
    )(h, S, deg, W2m, b2m.reshape(1, -1), W1uh, W1ua, b1u.reshape(1, -1),
      W2u, b2u.reshape(1, -1), ln_g.reshape(1, -1), ln_b.reshape(1, -1))


def kernel(h, edge_index, edge_attr, num_nodes, W1m, b1m, g1m, be1m, W2m, b2m,
           W1u, b1u, g1u, be1u, W2u, b2u, ln_g, ln_b):
    N, D = h.shape
    H = W1m.shape[1]

    s1m = g1m * (1.0 / jnp.sqrt(1.0 + EPS))
    Wsrc = W1m[:D] * s1m
    Wdst = W1m[D:2 * D] * s1m
    We = W1m[2 * D:] * s1m
    b1m_f = b1m * s1m + be1m
    s1u = g1u * (1.0 / jnp.sqrt(1.0 + EPS))
    W1uh = W1u[:D] * s1u
    W1ua = W1u[D:] * s1u
    b1u_f = b1u * s1u + be1u

    src = edge_index[0]
    dst = edge_index[1]

    C, A, B = _pre_all(h, Wsrc, Wdst, edge_attr, We, b1m_f)
    S = _edge_msg(src, dst, A, B, C)
    deg = _edge_deg(dst, N, H)
    return _node_post(h, S, deg, W2m, b2m, W1uh, W1ua, b1u_f, W2u, b2u, ln_g, ln_b)


# speedup vs baseline: 1.2808x; 1.0597x over previous
import functools

import jax
import jax.numpy as jnp
from jax import lax
from jax.experimental import pallas as pl
from jax.experimental.pallas import tpu as pltpu
from jax.experimental.pallas import tpu_sc as plsc

EPS = 1e-5

NC = 2
NS = 16
K = 64


def _pre_all(h, Ws, Wd, ea, We, b):
    N, D = h.shape
    E, ED = ea.shape
    H = We.shape[1]
    BE = 8000
    BN = 2000
    NBA = N // BN
    clamp = lambda i: (jnp.minimum(i, NBA - 1), 0)

    def body(ea_ref, we_ref, b_ref, h_ref, ws_ref, wd_ref, c_ref, a_ref, b2_ref):
        c_ref[...] = (
            jnp.dot(ea_ref[...], we_ref[...], preferred_element_type=jnp.float32)
            + b_ref[...]
        )
        hb = h_ref[...]
        a_ref[...] = jnp.dot(hb, ws_ref[...], preferred_element_type=jnp.float32)
        b2_ref[...] = jnp.dot(hb, wd_ref[...], preferred_element_type=jnp.float32)

    return pl.pallas_call(
        body,
        grid=(E // BE,),
        in_specs=[
            pl.BlockSpec((BE, ED), lambda i: (i, 0)),
            pl.BlockSpec((ED, H), lambda i: (0, 0)),
            pl.BlockSpec((1, H), lambda i: (0, 0)),
            pl.BlockSpec((BN, D), clamp),
            pl.BlockSpec((D, H), lambda i: (0, 0)),
            pl.BlockSpec((D, H), lambda i: (0, 0)),
        ],
        out_specs=[
            pl.BlockSpec((BE, H), lambda i: (i, 0)),
            pl.BlockSpec((BN, H), clamp),
            pl.BlockSpec((BN, H), clamp),
        ],
        out_shape=[
            jax.ShapeDtypeStruct((E, H), jnp.float32),
            jax.ShapeDtypeStruct((N, H), jnp.float32),
            jax.ShapeDtypeStruct((N, H), jnp.float32),
        ],
    )(ea, We, b.reshape(1, H), h, Ws, Wd)


def _edge_msg(ei, dst, A, B, C):
    N, H = A.shape
    E = dst.shape[0]
    TCH = E // K
    CW = TCH // (NC * NS)
    XTRA = TCH % (NC * NS)
    RPT = (N // NS) // 8 * 8
    TAIL = N - NS * RPT
    mesh = plsc.VectorSubcoreMesh(core_axis_name="c", subcore_axis_name="s")

    @functools.partial(
        pl.kernel,
        out_type=[jax.ShapeDtypeStruct((NC, N, H), jnp.float32)],
        mesh=mesh,
        scratch_types=[
            pltpu.VMEM((2, K), jnp.int32),
            pltpu.VMEM((K, H), jnp.float32),
            pltpu.VMEM((K, H), jnp.float32),
            pltpu.VMEM((K, H), jnp.float32),
            pltpu.VMEM((2, K), jnp.int32),
            pltpu.VMEM((K, H), jnp.float32),
            pltpu.VMEM((K, H), jnp.float32),
            pltpu.VMEM((K, H), jnp.float32),
            pltpu.VMEM_SHARED((N, H), jnp.float32),
            pltpu.SemaphoreType.DMA,
            pltpu.SemaphoreType.DMA,
            pltpu.SemaphoreType.DMA,
            pltpu.SemaphoreType.DMA,
            pltpu.SemaphoreType.DMA,
            pltpu.SemaphoreType.DMA,
        ],
    )
    def body(ei_hbm, a_hbm, b_hbm, c_hbm, s_out,
             ib0, ra0, rb0, rc0,
             ib1, ra1, rb1, rc1, s_sh,
             semA0, semB0, semC0, semA1, semB1, semC1):
        c = lax.axis_index("c")
        s = lax.axis_index("s")
        z16 = jnp.zeros((16,), jnp.float32)
        sets = [
            (ib0, ra0, rb0, rc0, semA0, semB0, semC0),
            (ib1, ra1, rb1, rc1, semA1, semB1, semC1),
        ]

        def zero_row(i, _):
            for j in range(H // 16):
                ra0[i, pl.ds(j * 16, 16)] = z16
            return 0

        lax.fori_loop(0, K, zero_row, 0)

        nb = s * RPT
        nfull = RPT // K

        def zero_copy(q, _):
            pltpu.sync_copy(ra0, s_sh.at[pl.ds(nb + q * K, K)])
            return 0

        lax.fori_loop(0, nfull, zero_copy, 0)
        rem = RPT - nfull * K
        if rem:
            pltpu.sync_copy(ra0.at[pl.ds(0, rem)], s_sh.at[pl.ds(nb + nfull * K, rem)])
        if TAIL:
            @pl.when(s == NS - 1)
            def _zero_tail():
                pltpu.sync_copy(ra0.at[pl.ds(0, TAIL)], s_sh.at[pl.ds(NS * RPT, TAIL)])

        plsc.subcore_barrier()

        w = c * NS + s
        my_start = w * CW + jnp.minimum(w, XTRA)
        my_n = CW + (w < XTRA).astype(jnp.int32)

        def start_gathers(b, g):
            ibb, rab, rbb, rcb, sA, sB, sC = sets[b]
            gc = my_start + g
            pltpu.sync_copy(ei_hbm.at[gc], ibb)
            pltpu.async_copy(a_hbm.at[ibb.at[0]], rab, sA)
            pltpu.async_copy(b_hbm.at[ibb.at[1]], rbb, sB)
            pltpu.async_copy(c_hbm.at[pl.ds(gc * K, K)], rcb, sC)

        def finish_chunk(b, g):
            ibb, rab, rbb, rcb, sA, sB, sC = sets[b]
            pltpu.make_async_copy(a_hbm.at[ibb.at[0]], rab, sA).wait()
            pltpu.make_async_copy(b_hbm.at[ibb.at[1]], rbb, sB).wait()
            pltpu.make_async_copy(c_hbm.at[pl.ds(0, K)], rcb, sC).wait()

            @plsc.parallel_loop(0, K, unroll=4)
            def row(i):
                for j in range(H // 16):
                    sl = pl.ds(j * 16, 16)
                    rab[i, sl] = jnp.maximum(rab[i, sl] + rbb[i, sl] + rcb[i, sl], 0.0)

            pltpu.sync_copy(rab, s_sh.at[ibb.at[1]], add=True)

            @pl.when(g + 2 < my_n)
            def _prefetch():
                start_gathers(b, g + 2)

        start_gathers(0, 0)
        start_gathers(1, 1)

        def pair(gp, _):
            finish_chunk(0, 2 * gp)
            finish_chunk(1, 2 * gp + 1)
            return 0

        lax.fori_loop(0, my_n // 2, pair, 0)

        @pl.when(my_n % 2 == 1)
        def _odd_tail():
            finish_chunk(0, my_n - 1)

        plsc.subcore_barrier()
        pltpu.sync_copy(s_sh.at[pl.ds(nb, RPT)], s_out.at[c, pl.ds(nb, RPT)])
        if TAIL:
            @pl.when(s == NS - 1)
            def _out_tail():
                pltpu.sync_copy(s_sh.at[pl.ds(NS * RPT, TAIL)],
                                s_out.at[c, pl.ds(NS * RPT, TAIL)])

    return body(ei, A, B, C)[0]


def _edge_deg(dst, N, H):
    E = dst.shape[0]
    KD = 80
    EW = E // (NC * NS)
    NCHUNK = EW // KD
    RPT = (N // NS) // 8 * 8
    TAIL = N - NS * RPT
    mesh = plsc.VectorSubcoreMesh(core_axis_name="c", subcore_axis_name="s")

    @functools.partial(
        pl.kernel,
        out_type=[jax.ShapeDtypeStruct((NC, N, H), jnp.float32)],
        mesh=mesh,
        scratch_types=[
            pltpu.VMEM((KD,), jnp.int32),
            pltpu.VMEM((KD,), jnp.int32),
            pltpu.VMEM((KD, H), jnp.float32),
            pltpu.VMEM_SHARED((N, H), jnp.float32),
            pltpu.SemaphoreType.DMA,
            pltpu.SemaphoreType.DMA,
        ],
    )
    def body(dst_hbm, deg_out, id0, id1, dval, d_sh, semS0, semS1):
        c = lax.axis_index("c")
        s = lax.axis_index("s")
        z16 = jnp.zeros((16,), jnp.float32)

        def zero_row(i, _):
            for j in range(H // 16):
                dval[i, pl.ds(j * 16, 16)] = z16
            return 0

        lax.fori_loop(0, KD, zero_row, 0)

        nb = s * RPT
        nfull = RPT // KD

        def zero_copy(q, _):
            pltpu.sync_copy(dval, d_sh.at[pl.ds(nb + q * KD, KD)])
            return 0

        lax.fori_loop(0, nfull, zero_copy, 0)
        rem = RPT - nfull * KD
        if rem:
            pltpu.sync_copy(dval.at[pl.ds(0, rem)], d_sh.at[pl.ds(nb + nfull * KD, rem)])
        if TAIL:
            @pl.when(s == NS - 1)
            def _zero_tail():
                pltpu.sync_copy(dval.at[pl.ds(0, TAIL)], d_sh.at[pl.ds(NS * RPT, TAIL)])

        e0 = jnp.where(lax.iota(jnp.int32, 16) == 0, 1.0, 0.0).astype(jnp.float32)

        def ones_row(i, _):
            dval[i, pl.ds(0, 16)] = e0
            return 0

        lax.fori_loop(0, KD, ones_row, 0)

        plsc.subcore_barrier()

        base_e = (c * NS + s) * EW
        sets = [(id0, semS0), (id1, semS1)]

        def half(b, gp, g):
            idb, sS = sets[b]

            @pl.when(gp > 0)
            def _drain():
                pltpu.make_async_copy(dval, d_sh.at[idb], sS).wait()

            pltpu.sync_copy(dst_hbm.at[pl.ds(base_e + g * KD, KD)], idb)
            pltpu.async_copy(dval, d_sh.at[idb], sS, add=True)

        def pair(gp, _):
            half(0, gp, 2 * gp)
            half(1, gp, 2 * gp + 1)
            return 0

        NPAIR = NCHUNK // 2
        lax.fori_loop(0, NPAIR, pair, 0)
        if NCHUNK % 2:
            half(0, NPAIR, NCHUNK - 1)
        pltpu.make_async_copy(dval, d_sh.at[id0], semS0).wait()
        if NCHUNK > 1:
            pltpu.make_async_copy(dval, d_sh.at[id1], semS1).wait()

        plsc.subcore_barrier()
        pltpu.sync_copy(d_sh.at[pl.ds(nb, RPT)], deg_out.at[c, pl.ds(nb, RPT)])
        if TAIL:
            @pl.when(s == NS - 1)
            def _out_tail():
                pltpu.sync_copy(d_sh.at[pl.ds(NS * RPT, TAIL)],
                                deg_out.at[c, pl.ds(NS * RPT, TAIL)])

    return body(dst)[0]


def _node_post(h, S, deg, W2m, b2m, W1uh, W1ua, b1u, W2u, b2u, ln_g, ln_b):
    N, D = h.shape
    H = W2m.shape[0]
    BN = 2000

    def body(h_ref, s_ref, d_ref, w2m_ref, b2m_ref, w1uh_ref, w1ua_ref,
             b1u_ref, w2u_ref, b2u_ref, lng_ref, lnb_ref, out_ref):
        hb = h_ref[...]
        Sb = s_ref[0] + s_ref[1]
        degb = d_ref[0, :, 0:1] + d_ref[1, :, 0:1]
        agg = (
            jnp.dot(Sb, w2m_ref[...], preferred_element_type=jnp.float32)
            + degb * b2m_ref[...]
        )
        y = (
            jnp.dot(hb, w1uh_ref[...], preferred_element_type=jnp.float32)
            + jnp.dot(agg, w1ua_ref[...], preferred_element_type=jnp.float32)
            + b1u_ref[...]
        )
        y = jnp.maximum(y, 0.0)
        hn = jnp.dot(y, w2u_ref[...], preferred_element_type=jnp.float32) + b2u_ref[...]
        z = hb + hn
        mean = jnp.mean(z, axis=1, keepdims=True)
        zc = z - mean
        var = jnp.mean(zc * zc, axis=1, keepdims=True)
        out_ref[...] = zc * lax.rsqrt(var + EPS) * lng_ref[...] + lnb_ref[...]

    full = lambda r, c: pl.BlockSpec((r, c), lambda i: (0, 0))
    return pl.pallas_call(
        body,
        grid=(N // BN,),
        in_specs=[
            pl.BlockSpec((BN, D), lambda i: (i, 0)),
            pl.BlockSpec((NC, BN, H), lambda i: (0, i, 0)),
            pl.BlockSpec((NC, BN, H), lambda i: (0, i, 0)),
            full(H, D), full(1, D), full(D, H), full(H, H), full(1, H),
            full(H, D), full(1, D), full(1, D), full(1, D),
        ],
        out_specs=pl.BlockSpec((BN, D), lambda i: (i, 0)),
        out_shape=jax.ShapeDtypeStruct((N, D), jnp.float32),
    )(h, S, deg, W2m, b2m.reshape(1, -1), W1uh, W1ua, b1u.reshape(1, -1),
      W2u, b2u.reshape(1, -1), ln_g.reshape(1, -1), ln_b.reshape(1, -1))


def kernel(h, edge_index, edge_attr, num_nodes, W1m, b1m, g1m, be1m, W2m, b2m,
           W1u, b1u, g1u, be1u, W2u, b2u, ln_g, ln_b):
    N, D = h.shape
    H = W1m.shape[1]

    s1m = g1m * (1.0 / jnp.sqrt(1.0 + EPS))
    Wsrc = W1m[:D] * s1m
    Wdst = W1m[D:2 * D] * s1m
    We = W1m[2 * D:] * s1m
    b1m_f = b1m * s1m + be1m
    s1u = g1u * (1.0 / jnp.sqrt(1.0 + EPS))
    W1uh = W1u[:D] * s1u
    W1ua = W1u[D:] * s1u
    b1u_f = b1u * s1u + be1u

    src = edge_index[0]
    dst = edge_index[1]
    TCH = edge_index.shape[1] // K
    ei = jnp.stack([src.reshape(TCH, K), dst.reshape(TCH, K)], axis=1)

    C, A, B = _pre_all(h, Wsrc, Wdst, edge_attr, We, b1m_f)
    S = _edge_msg(ei, dst, A, B, C)
    deg = _edge_deg(dst, N, H)
    return _node_post(h, S, deg, W2m, b2m, W1uh, W1ua, b1u_f, W2u, b2u, ln_g, ln_b)
